# Initial kernel scaffold; baseline (speedup 1.0000x reference)
import jax
import jax.numpy as jnp
from jax.experimental import pallas as pl


def kernel(x, edge_index, gate, forward_level, forward_index, Ws, Wt, W_hs, b_hs, aggr_W, aggr_b, gru_Wih, gru_bih, gru_Whh, gru_bhh):
    raise NotImplementedError("write your pallas kernel here")


# speedup vs baseline: 9.7704x; 9.7704x over previous
import functools

import jax
import jax.numpy as jnp
from jax import lax
from jax.experimental import pallas as pl
from jax.experimental.pallas import tpu as pltpu
from jax.experimental.pallas import tpu_sc as plsc

N = 10000
E = 160000
D = 128

NC = 2
NS = 16
NW = NC * NS
CH = 128
PT = 5120
EP = PT * NW
PTA = EP // NS
NACC = 10240
NST = 640
NSTL = N - 15 * NST
ZR = NACC // NS
NB = 1000


def _encode_body(x_ref, ws_ref, wt_ref, xs_ref, xt_ref):
    col = x_ref[:, 1:2]
    oh = (col == lax.broadcasted_iota(jnp.int32, (NB, 8), 1)).astype(jnp.float32)
    xs_ref[...] = jnp.dot(oh, ws_ref[...], preferred_element_type=jnp.float32)
    xt_ref[...] = jnp.dot(oh, wt_ref[...], preferred_element_type=jnp.float32)


def _encode(x, ws_p, wt_p):
    return pl.pallas_call(
        _encode_body,
        grid=(N // NB,),
        in_specs=[
            pl.BlockSpec((NB, 2), lambda i: (i, 0)),
            pl.BlockSpec((8, D), lambda i: (0, 0)),
            pl.BlockSpec((8, D), lambda i: (0, 0)),
        ],
        out_specs=[
            pl.BlockSpec((NB, D), lambda i: (i, 0)),
            pl.BlockSpec((NB, D), lambda i: (i, 0)),
        ],
        out_shape=[
            jax.ShapeDtypeStruct((N, D), jnp.float32),
            jax.ShapeDtypeStruct((N, D), jnp.float32),
        ],
    )(x, ws_p, wt_p)


def _mix_body(xs_ref, xt_ref, ags_ref, agt_ref, wa_ref, wb_ref, b_ref,
              at_ref, ab_ref, hs_ref, mn_ref):
    s = jax.nn.relu(xs_ref[...] + ags_ref[...])
    t = jax.nn.relu(xt_ref[...] + agt_ref[...])
    hsb = (jnp.dot(s, wa_ref[...], preferred_element_type=jnp.float32)
           + jnp.dot(t, wb_ref[...], preferred_element_type=jnp.float32)
           + b_ref[...])
    hs_ref[...] = hsb
    for g in range(4):
        mn_ref[g] = jax.nn.relu(
            jnp.dot(hsb, at_ref[g], preferred_element_type=jnp.float32)
            + ab_ref[g])


def _mix(xs, xt, ags, agt, wa, wb, b2, at, ab2):
    return pl.pallas_call(
        _mix_body,
        grid=(N // NB,),
        in_specs=[
            pl.BlockSpec((NB, D), lambda i: (i, 0)),
            pl.BlockSpec((NB, D), lambda i: (i, 0)),
            pl.BlockSpec((NB, D), lambda i: (i, 0)),
            pl.BlockSpec((NB, D), lambda i: (i, 0)),
            pl.BlockSpec((D, D), lambda i: (0, 0)),
            pl.BlockSpec((D, D), lambda i: (0, 0)),
            pl.BlockSpec((1, D), lambda i: (0, 0)),
            pl.BlockSpec((4, D, D), lambda i: (0, 0, 0)),
            pl.BlockSpec((4, 1, D), lambda i: (0, 0, 0)),
        ],
        out_specs=[
            pl.BlockSpec((NB, D), lambda i: (i, 0)),
            pl.BlockSpec((4, NB, D), lambda i: (0, i, 0)),
        ],
        out_shape=[
            jax.ShapeDtypeStruct((N, D), jnp.float32),
            jax.ShapeDtypeStruct((4, N, D), jnp.float32),
        ],
    )(xs, xt, ags, agt, wa, wb, b2, at, ab2)


def _gru_body(level, with_mn, msgp_ref, hs_ref, hf_ref, gate_ref, lv_ref,
              wih_ref, bih_ref, whh_ref, bhh_ref, at_ref, abm_ref, ab_ref,
              hfo_ref, *mn_out):
    m = msgp_ref[0] + msgp_ref[1]
    g = gate_ref[...]
    gi = jnp.where(g == 2, 0, jnp.where(g == 3, 1,
         jnp.where(g == 4, 2, jnp.where(g == 1, 3, -1))))
    oh = (gi == lax.broadcasted_iota(jnp.int32, (NB, 4), 1)).astype(jnp.float32)
    hf = hf_ref[...]
    gi_lin = jnp.zeros((NB, 3 * D), jnp.float32)
    gh_lin = jnp.zeros((NB, 3 * D), jnp.float32)
    for gg in range(4):
        ohg = oh[:, gg:gg + 1]
        gi_lin = gi_lin + jnp.dot(m * ohg, wih_ref[gg],
                                  preferred_element_type=jnp.float32) + ohg * bih_ref[gg]
        gh_lin = gh_lin + jnp.dot(hf * ohg, whh_ref[gg],
                                  preferred_element_type=jnp.float32) + ohg * bhh_ref[gg]
    r = jax.nn.sigmoid(gi_lin[:, :D] + gh_lin[:, :D])
    z = jax.nn.sigmoid(gi_lin[:, D:2 * D] + gh_lin[:, D:2 * D])
    ng = jnp.tanh(gi_lin[:, 2 * D:] + r * gh_lin[:, 2 * D:])
    h_new = (1.0 - z) * ng + z * hf
    mask = (lv_ref[...] == level) & (gi >= 0)
    hf_new = jnp.where(mask, h_new, hf)
    hfo_ref[...] = hf_new
    if with_mn:
        hsb = hs_ref[...]
        for gg in range(4):
            mn_out[0][gg] = jax.nn.relu(
                jnp.dot(hsb, at_ref[gg], preferred_element_type=jnp.float32)
                + jnp.dot(hf_new, abm_ref[gg], preferred_element_type=jnp.float32)
                + ab_ref[gg])


def _gru(level, with_mn, msgp, hs, hf, gate2, lv2, wih, bih2, whh, bhh2,
         at, abm, ab2):
    out_specs = [pl.BlockSpec((NB, D), lambda i: (i, 0))]
    out_shape = [jax.ShapeDtypeStruct((N, D), jnp.float32)]
    if with_mn:
        out_specs.append(pl.BlockSpec((4, NB, D), lambda i: (0, i, 0)))
        out_shape.append(jax.ShapeDtypeStruct((4, N, D), jnp.float32))
    return pl.pallas_call(
        functools.partial(_gru_body, level, with_mn),
        grid=(N // NB,),
        in_specs=[
            pl.BlockSpec((2, NB, D), lambda i: (0, i, 0)),
            pl.BlockSpec((NB, D), lambda i: (i, 0)),
            pl.BlockSpec((NB, D), lambda i: (i, 0)),
            pl.BlockSpec((NB, 1), lambda i: (i, 0)),
            pl.BlockSpec((NB, 1), lambda i: (i, 0)),
            pl.BlockSpec((4, D, 3 * D), lambda i: (0, 0, 0)),
            pl.BlockSpec((4, 1, 3 * D), lambda i: (0, 0, 0)),
            pl.BlockSpec((4, D, 3 * D), lambda i: (0, 0, 0)),
            pl.BlockSpec((4, 1, 3 * D), lambda i: (0, 0, 0)),
            pl.BlockSpec((4, D, D), lambda i: (0, 0, 0)),
            pl.BlockSpec((4, D, D), lambda i: (0, 0, 0)),
            pl.BlockSpec((4, 1, D), lambda i: (0, 0, 0)),
        ],
        out_specs=out_specs,
        out_shape=out_shape,
    )(msgp, hs, hf, gate2, lv2, wih, bih2, whh, bhh2, at, abm, ab2)


_MESH = plsc.VectorSubcoreMesh(core_axis_name="c", subcore_axis_name="s")


def _copy_out_stripe(ts, accum, out_hbm):
    @pl.when(ts < 15)
    def _():
        off = pl.multiple_of(ts * NST, 8)
        pltpu.sync_copy(accum.at[pl.ds(off, NST)], out_hbm.at[pl.ds(off, NST)])

    @pl.when(ts == 15)
    def _():
        pltpu.sync_copy(accum.at[pl.ds(15 * NST, NSTL)],
                        out_hbm.at[pl.ds(15 * NST, NSTL)])


def _fix_scatter_idx(sidx_v, raw_ref, off, trash, valid_limit):
    for j in range(CH // 16):
        eidx = off + j * 16 + lax.iota(jnp.int32, 16)
        raw = raw_ref[pl.ds(j * 16, 16)]
        sidx_v[pl.ds(j * 16, 16)] = jnp.where(eidx < valid_limit, raw, trash)


def _segsum_kernel(xs_hbm, xt_hbm, src_hbm, dst_hbm, gate_hbm, lvl_hbm, z_hbm,
                   aggs_hbm, aggt_hbm, gie_hbm, lve_hbm,
                   gidx_v, sraw_v, sidx_v, rows_v, mg_v, ml_v, accum, sem):
    c = lax.axis_index("c")
    ts = lax.axis_index("s")
    wid = ts * NC + c
    trash = N + ts

    pltpu.sync_copy(z_hbm, accum.at[pl.ds(ts * ZR, ZR)])
    plsc.subcore_barrier()

    def meta_chunk(ci, _):
        off = pl.multiple_of(wid * PT + ci * CH, CH)
        pltpu.sync_copy(dst_hbm.at[pl.ds(off, CH)], gidx_v)
        pltpu.async_copy(gate_hbm.at[gidx_v], mg_v, sem).wait()
        pltpu.async_copy(lvl_hbm.at[gidx_v], ml_v, sem).wait()
        for j in range(CH // 16):
            sl = pl.ds(j * 16, 16)
            eidx = off + j * 16 + lax.iota(jnp.int32, 16)
            g = mg_v[sl]
            gi = jnp.where(g == 2, 0, jnp.where(g == 3, 1,
                 jnp.where(g == 4, 2, jnp.where(g == 1, 3, -1))))
            lv = jnp.where(eidx < E, ml_v[sl], -1)
            mg_v[sl] = gi
            ml_v[sl] = lv
        pltpu.sync_copy(mg_v, gie_hbm.at[pl.ds(off, CH)])
        pltpu.sync_copy(ml_v, lve_hbm.at[pl.ds(off, CH)])
        return 0

    lax.fori_loop(0, PT // CH, meta_chunk, 0)

    def agg_chunk(tab_hbm, g_src, s_src, ci):
        off = pl.multiple_of(ts * PTA + ci * CH, CH)
        pltpu.sync_copy(g_src.at[pl.ds(off, CH)], gidx_v)
        pltpu.sync_copy(s_src.at[pl.ds(off, CH)], sraw_v)
        _fix_scatter_idx(sidx_v, sraw_v, off, trash, E)
        pltpu.async_copy(tab_hbm.at[gidx_v], rows_v, sem).wait()
        pltpu.sync_copy(rows_v, accum.at[sidx_v], add=True)

    @pl.when(c == 0)
    def _():
        lax.fori_loop(0, PTA // CH,
                      lambda ci, _: (agg_chunk(xs_hbm, src_hbm, dst_hbm, ci), 0)[1], 0)

    @pl.when(c == 1)
    def _():
        lax.fori_loop(0, PTA // CH,
                      lambda ci, _: (agg_chunk(xt_hbm, dst_hbm, src_hbm, ci), 0)[1], 0)

    plsc.subcore_barrier()

    @pl.when(c == 0)
    def _():
        _copy_out_stripe(ts, accum, aggs_hbm)

    @pl.when(c == 1)
    def _():
        _copy_out_stripe(ts, accum, aggt_hbm)


_segsum = pl.kernel(
    _segsum_kernel, mesh=_MESH,
    out_type=[
        jax.ShapeDtypeStruct((N, D), jnp.float32),
        jax.ShapeDtypeStruct((N, D), jnp.float32),
        jax.ShapeDtypeStruct((EP,), jnp.int32),
        jax.ShapeDtypeStruct((EP,), jnp.int32),
    ],
    scratch_types=[
        pltpu.VMEM((CH,), jnp.int32),
        pltpu.VMEM((CH,), jnp.int32),
        pltpu.VMEM((CH,), jnp.int32),
        pltpu.VMEM((CH, D), jnp.float32),
        pltpu.VMEM((CH,), jnp.int32),
        pltpu.VMEM((CH,), jnp.int32),
        pltpu.VMEM_SHARED((NACC, D), jnp.float32),
        pltpu.SemaphoreType.DMA,
    ],
)


def _msg_kernel(level, mn_hbm, src_hbm, dst_hbm, gie_hbm, lve_hbm, z_hbm,
                msg_hbm, gidx_v, sidx_v, sv_v, dv_v, gv_v, lv_v, rows_v,
                accum, sem):
    c = lax.axis_index("c")
    ts = lax.axis_index("s")
    wid = ts * NC + c
    trash = N + ts

    pltpu.sync_copy(z_hbm, accum.at[pl.ds(ts * ZR, ZR)])
    plsc.subcore_barrier()

    def chunk(ci, _):
        off = pl.multiple_of(wid * PT + ci * CH, CH)
        pltpu.sync_copy(src_hbm.at[pl.ds(off, CH)], sv_v)
        pltpu.sync_copy(dst_hbm.at[pl.ds(off, CH)], dv_v)
        pltpu.sync_copy(gie_hbm.at[pl.ds(off, CH)], gv_v)
        pltpu.sync_copy(lve_hbm.at[pl.ds(off, CH)], lv_v)
        for j in range(CH // 16):
            sl = pl.ds(j * 16, 16)
            g = gv_v[sl]
            active = (lv_v[sl] == level) & (g >= 0)
            gidx_v[sl] = jnp.maximum(g, 0) * N + sv_v[sl]
            sidx_v[sl] = jnp.where(active, dv_v[sl], trash)
        pltpu.async_copy(mn_hbm.at[gidx_v], rows_v, sem).wait()
        pltpu.sync_copy(rows_v, accum.at[sidx_v], add=True)
        return 0

    lax.fori_loop(0, PT // CH, chunk, 0)
    plsc.subcore_barrier()
    _copy_out_stripe(ts, accum, msg_hbm.at[c])


def _msg(level):
    return pl.kernel(
        functools.partial(_msg_kernel, level), mesh=_MESH,
        out_type=jax.ShapeDtypeStruct((2, N, D), jnp.float32),
        scratch_types=[
            pltpu.VMEM((CH,), jnp.int32),
            pltpu.VMEM((CH,), jnp.int32),
            pltpu.VMEM((CH,), jnp.int32),
            pltpu.VMEM((CH,), jnp.int32),
            pltpu.VMEM((CH,), jnp.int32),
            pltpu.VMEM((CH,), jnp.int32),
            pltpu.VMEM((CH, D), jnp.float32),
            pltpu.VMEM_SHARED((NACC, D), jnp.float32),
            pltpu.SemaphoreType.DMA,
        ],
    )


def kernel(x, edge_index, gate, forward_level, forward_index, Ws, Wt, W_hs,
           b_hs, aggr_W, aggr_b, gru_Wih, gru_bih, gru_Whh, gru_bhh):
    f32 = jnp.float32
    src = jnp.pad(edge_index[0].astype(jnp.int32), (0, EP - E))
    dst = jnp.pad(edge_index[1].astype(jnp.int32), (0, EP - E))
    gate1 = gate[:, 0].astype(jnp.int32)
    lvl1 = forward_level.astype(jnp.int32)
    ws_p = jnp.pad(Ws.astype(f32), ((0, 2), (0, 0)))
    wt_p = jnp.pad(Wt.astype(f32), ((0, 2), (0, 0)))
    wa = W_hs[:D].astype(f32)
    wb = W_hs[D:].astype(f32)
    b2 = b_hs.reshape(1, D).astype(f32)
    at = aggr_W[:, :D, :].astype(f32)
    abm = aggr_W[:, D:, :].astype(f32)
    ab2 = aggr_b.reshape(4, 1, D).astype(f32)
    bih2 = gru_bih.reshape(4, 1, 3 * D).astype(f32)
    bhh2 = gru_bhh.reshape(4, 1, 3 * D).astype(f32)
    wih = gru_Wih.astype(f32)
    whh = gru_Whh.astype(f32)
    gate2 = gate1.reshape(N, 1)
    lv2 = lvl1.reshape(N, 1)
    zrows = jnp.zeros((ZR, D), f32)

    xs, xt = _encode(x.astype(jnp.int32), ws_p, wt_p)
    ags, agt, gie, lve = _segsum(xs, xt, src, dst, gate1, lvl1, zrows)
    hs, mn = _mix(xs, xt, ags, agt, wa, wb, b2, at, ab2)
    hf = jnp.zeros((N, D), f32)
    for level in range(1, 4):
        msgp = _msg(level)(mn.reshape(4 * N, D), src, dst, gie, lve, zrows)
        outs = _gru(level, level < 3, msgp, hs, hf, gate2, lv2, wih, bih2,
                    whh, bhh2, at, abm, ab2)
        if level < 3:
            hf, mn = outs
        else:
            hf, = outs
    return hs, hf


# speedup vs baseline: 12.1358x; 1.2421x over previous
import functools

import jax
import jax.numpy as jnp
from jax import lax
from jax.experimental import pallas as pl
from jax.experimental.pallas import tpu as pltpu
from jax.experimental.pallas import tpu_sc as plsc

N = 10000
E = 160000
D = 128

NC = 2
NS = 16
NW = NC * NS
CH = 128
PT = 5120
EP = PT * NW
PTA = EP // NS
NACC = 10240
NST = 640
NSTL = N - 15 * NST
ZR = NACC // NS
NB = 1000
R = 2


def _encode_body(x_ref, g_ref, l_ref, ws_ref, wt_ref, xs_ref, xt_ref, pk_ref):
    col = x_ref[:, 1:2]
    oh = (col == lax.broadcasted_iota(jnp.int32, (NB, 8), 1)).astype(jnp.float32)
    xs_ref[...] = jnp.dot(oh, ws_ref[...], preferred_element_type=jnp.float32)
    xt_ref[...] = jnp.dot(oh, wt_ref[...], preferred_element_type=jnp.float32)
    pk_ref[...] = g_ref[...] + 8 * l_ref[...]


def _encode(x, gate2, lv2, ws_p, wt_p):
    return pl.pallas_call(
        _encode_body,
        grid=(N // NB,),
        in_specs=[
            pl.BlockSpec((NB, 2), lambda i: (i, 0)),
            pl.BlockSpec((NB, 1), lambda i: (i, 0)),
            pl.BlockSpec((NB, 1), lambda i: (i, 0)),
            pl.BlockSpec((8, D), lambda i: (0, 0)),
            pl.BlockSpec((8, D), lambda i: (0, 0)),
        ],
        out_specs=[
            pl.BlockSpec((NB, D), lambda i: (i, 0)),
            pl.BlockSpec((NB, D), lambda i: (i, 0)),
            pl.BlockSpec((NB, 1), lambda i: (i, 0)),
        ],
        out_shape=[
            jax.ShapeDtypeStruct((N, D), jnp.float32),
            jax.ShapeDtypeStruct((N, D), jnp.float32),
            jax.ShapeDtypeStruct((N, 1), jnp.int32),
        ],
    )(x, gate2, lv2, ws_p, wt_p)


def _mix_body(xs_ref, xt_ref, ags_ref, agt_ref, wa_ref, wb_ref, b_ref,
              at_ref, ab_ref, hs_ref, mn_ref):
    s = jax.nn.relu(xs_ref[...] + ags_ref[...])
    t = jax.nn.relu(xt_ref[...] + agt_ref[...])
    hsb = (jnp.dot(s, wa_ref[...], preferred_element_type=jnp.float32)
           + jnp.dot(t, wb_ref[...], preferred_element_type=jnp.float32)
           + b_ref[...])
    hs_ref[...] = hsb
    for g in range(4):
        mn_ref[g] = jax.nn.relu(
            jnp.dot(hsb, at_ref[g], preferred_element_type=jnp.float32)
            + ab_ref[g])


def _mix(xs, xt, ags, agt, wa, wb, b2, at, ab2):
    return pl.pallas_call(
        _mix_body,
        grid=(N // NB,),
        in_specs=[
            pl.BlockSpec((NB, D), lambda i: (i, 0)),
            pl.BlockSpec((NB, D), lambda i: (i, 0)),
            pl.BlockSpec((NB, D), lambda i: (i, 0)),
            pl.BlockSpec((NB, D), lambda i: (i, 0)),
            pl.BlockSpec((D, D), lambda i: (0, 0)),
            pl.BlockSpec((D, D), lambda i: (0, 0)),
            pl.BlockSpec((1, D), lambda i: (0, 0)),
            pl.BlockSpec((4, D, D), lambda i: (0, 0, 0)),
            pl.BlockSpec((4, 1, D), lambda i: (0, 0, 0)),
        ],
        out_specs=[
            pl.BlockSpec((NB, D), lambda i: (i, 0)),
            pl.BlockSpec((4, NB, D), lambda i: (0, i, 0)),
        ],
        out_shape=[
            jax.ShapeDtypeStruct((N, D), jnp.float32),
            jax.ShapeDtypeStruct((4, N, D), jnp.float32),
        ],
    )(xs, xt, ags, agt, wa, wb, b2, at, ab2)


def _gru_body(level, with_mn, msgp_ref, hs_ref, hf_ref, gate_ref, lv_ref,
              wih_ref, bih_ref, whh_ref, bhh_ref, at_ref, abm_ref, ab_ref,
              hfo_ref, *mn_out):
    m = msgp_ref[0] + msgp_ref[1]
    g = gate_ref[...]
    gi = jnp.where(g == 2, 0, jnp.where(g == 3, 1,
         jnp.where(g == 4, 2, jnp.where(g == 1, 3, -1))))
    oh = (gi == lax.broadcasted_iota(jnp.int32, (NB, 4), 1)).astype(jnp.float32)
    hf = hf_ref[...]
    gi_lin = jnp.zeros((NB, 3 * D), jnp.float32)
    gh_lin = jnp.zeros((NB, 3 * D), jnp.float32)
    for gg in range(4):
        ohg = oh[:, gg:gg + 1]
        gi_lin = gi_lin + jnp.dot(m * ohg, wih_ref[gg],
                                  preferred_element_type=jnp.float32) + ohg * bih_ref[gg]
        gh_lin = gh_lin + jnp.dot(hf * ohg, whh_ref[gg],
                                  preferred_element_type=jnp.float32) + ohg * bhh_ref[gg]
    r = jax.nn.sigmoid(gi_lin[:, :D] + gh_lin[:, :D])
    z = jax.nn.sigmoid(gi_lin[:, D:2 * D] + gh_lin[:, D:2 * D])
    ng = jnp.tanh(gi_lin[:, 2 * D:] + r * gh_lin[:, 2 * D:])
    h_new = (1.0 - z) * ng + z * hf
    mask = (lv_ref[...] == level) & (gi >= 0)
    hf_new = jnp.where(mask, h_new, hf)
    hfo_ref[...] = hf_new
    if with_mn:
        hsb = hs_ref[...]
        for gg in range(4):
            mn_out[0][gg] = jax.nn.relu(
                jnp.dot(hsb, at_ref[gg], preferred_element_type=jnp.float32)
                + jnp.dot(hf_new, abm_ref[gg], preferred_element_type=jnp.float32)
                + ab_ref[gg])


def _gru(level, with_mn, msgp, hs, hf, gate2, lv2, wih, bih2, whh, bhh2,
         at, abm, ab2):
    out_specs = [pl.BlockSpec((NB, D), lambda i: (i, 0))]
    out_shape = [jax.ShapeDtypeStruct((N, D), jnp.float32)]
    if with_mn:
        out_specs.append(pl.BlockSpec((4, NB, D), lambda i: (0, i, 0)))
        out_shape.append(jax.ShapeDtypeStruct((4, N, D), jnp.float32))
    return pl.pallas_call(
        functools.partial(_gru_body, level, with_mn),
        grid=(N // NB,),
        in_specs=[
            pl.BlockSpec((2, NB, D), lambda i: (0, i, 0)),
            pl.BlockSpec((NB, D), lambda i: (i, 0)),
            pl.BlockSpec((NB, D), lambda i: (i, 0)),
            pl.BlockSpec((NB, 1), lambda i: (i, 0)),
            pl.BlockSpec((NB, 1), lambda i: (i, 0)),
            pl.BlockSpec((4, D, 3 * D), lambda i: (0, 0, 0)),
            pl.BlockSpec((4, 1, 3 * D), lambda i: (0, 0, 0)),
            pl.BlockSpec((4, D, 3 * D), lambda i: (0, 0, 0)),
            pl.BlockSpec((4, 1, 3 * D), lambda i: (0, 0, 0)),
            pl.BlockSpec((4, D, D), lambda i: (0, 0, 0)),
            pl.BlockSpec((4, D, D), lambda i: (0, 0, 0)),
            pl.BlockSpec((4, 1, D), lambda i: (0, 0, 0)),
        ],
        out_specs=out_specs,
        out_shape=out_shape,
    )(msgp, hs, hf, gate2, lv2, wih, bih2, whh, bhh2, at, abm, ab2)


_MESH = plsc.VectorSubcoreMesh(core_axis_name="c", subcore_axis_name="s")


def _copy_out_stripe(ts, accum, out_hbm):
    @pl.when(ts < 15)
    def _():
        off = pl.multiple_of(ts * NST, 8)
        pltpu.sync_copy(accum.at[pl.ds(off, NST)], out_hbm.at[pl.ds(off, NST)])

    @pl.when(ts == 15)
    def _():
        pltpu.sync_copy(accum.at[pl.ds(15 * NST, NSTL)],
                        out_hbm.at[pl.ds(15 * NST, NSTL)])


def _segsum_kernel(xs_hbm, xt_hbm, src_hbm, dst_hbm, pk_hbm, z_hbm,
                   aggs_hbm, aggt_hbm, midx_hbm,
                   db, sb, pv, mo_v, gb, rb, si, rows, accum, sems, semm):
    c = lax.axis_index("c")
    ts = lax.axis_index("s")
    wid = ts * NC + c

    pltpu.sync_copy(z_hbm, accum.at[pl.ds(ts * ZR, ZR)])

    M1 = PT // CH

    def meta_group(k, _):
        base = pl.multiple_of(wid * PT + k * (R * CH), CH)
        cps = []
        for r in range(R):
            off = base + r * CH
            pltpu.sync_copy(dst_hbm.at[pl.ds(off, CH)], db[r])
            pltpu.sync_copy(src_hbm.at[pl.ds(off, CH)], sb[r])
            cps.append(pltpu.async_copy(pk_hbm.at[db[r]], pv[r], semm[r]))
        for r in range(R):
            off = base + r * CH
            cps[r].wait()
            for j in range(CH // 16):
                sl = pl.ds(j * 16, 16)
                eidx = off + j * 16 + lax.iota(jnp.int32, 16)
                p = pv[r][sl]
                g = p & 7
                lv = jnp.where(eidx < E, p >> 3, -1)
                gi = jnp.where(g == 2, 0, jnp.where(g == 3, 1,
                     jnp.where(g == 4, 2, jnp.where(g == 1, 3, -1))))
                tr = N + (eidx & 15)
                d16 = db[r][sl]
                mo_v[sl] = jnp.maximum(gi, 0) * N + sb[r][sl]
                for l in range(1, 4):
                    mo_v[pl.ds(l * CH + j * 16, 16)] = jnp.where(
                        (lv == l) & (gi >= 0), d16, tr)
            pltpu.sync_copy(mo_v, midx_hbm.at[pl.ds(off * 4, 4 * CH)])
        return 0

    lax.fori_loop(0, M1 // R, meta_group, 0)
    plsc.subcore_barrier()

    M2 = PTA // CH

    def agg_group(tab_hbm, g_src, s_src, k):
        base = pl.multiple_of(ts * PTA + k * (R * CH), CH)
        cps = []
        for r in range(R):
            off = base + r * CH
            pltpu.sync_copy(g_src.at[pl.ds(off, CH)], gb[r])
            pltpu.sync_copy(s_src.at[pl.ds(off, CH)], rb[r])
            cps.append(pltpu.async_copy(tab_hbm.at[gb[r]], rows[r], sems[r]))
        for r in range(R):
            off = base + r * CH
            for j in range(CH // 16):
                sl = pl.ds(j * 16, 16)
                eidx = off + j * 16 + lax.iota(jnp.int32, 16)
                si[r][sl] = jnp.where(eidx < E, rb[r][sl], N + ts)
            cps[r].wait()
            pltpu.sync_copy(rows[r], accum.at[si[r]], add=True)
        return 0

    @pl.when(c == 0)
    def _():
        lax.fori_loop(0, M2 // R,
                      lambda k, _: agg_group(xs_hbm, src_hbm, dst_hbm, k), 0)

    @pl.when(c == 1)
    def _():
        lax.fori_loop(0, M2 // R,
                      lambda k, _: agg_group(xt_hbm, dst_hbm, src_hbm, k), 0)

    plsc.subcore_barrier()

    @pl.when(c == 0)
    def _():
        _copy_out_stripe(ts, accum, aggs_hbm)

    @pl.when(c == 1)
    def _():
        _copy_out_stripe(ts, accum, aggt_hbm)


_segsum = pl.kernel(
    _segsum_kernel, mesh=_MESH,
    out_type=[
        jax.ShapeDtypeStruct((N, D), jnp.float32),
        jax.ShapeDtypeStruct((N, D), jnp.float32),
        jax.ShapeDtypeStruct((4 * EP,), jnp.int32),
    ],
    scratch_types=[
        [pltpu.VMEM((CH,), jnp.int32) for _ in range(R)],
        [pltpu.VMEM((CH,), jnp.int32) for _ in range(R)],
        [pltpu.VMEM((CH,), jnp.int32) for _ in range(R)],
        pltpu.VMEM((4 * CH,), jnp.int32),
        [pltpu.VMEM((CH,), jnp.int32) for _ in range(R)],
        [pltpu.VMEM((CH,), jnp.int32) for _ in range(R)],
        [pltpu.VMEM((CH,), jnp.int32) for _ in range(R)],
        [pltpu.VMEM((CH, D), jnp.float32) for _ in range(R)],
        pltpu.VMEM_SHARED((NACC, D), jnp.float32),
        [pltpu.SemaphoreType.DMA for _ in range(R)],
        [pltpu.SemaphoreType.DMA for _ in range(R)],
    ],
)


def _msg_kernel(level, mn_hbm, midx_hbm, z_hbm, msg_hbm,
                mo, gi_v, si, rows, accum, sems):
    c = lax.axis_index("c")
    ts = lax.axis_index("s")
    wid = ts * NC + c

    pltpu.sync_copy(z_hbm, accum.at[pl.ds(ts * ZR, ZR)])
    plsc.subcore_barrier()

    M = PT // CH

    def group(k, _):
        base = pl.multiple_of((wid * PT + k * (R * CH)) * 4, 4 * CH)
        cps = []
        for r in range(R):
            pltpu.sync_copy(midx_hbm.at[pl.ds(base + r * (4 * CH), 4 * CH)], mo[r])
            for j in range(CH // 16):
                gi_v[r][pl.ds(j * 16, 16)] = mo[r][pl.ds(j * 16, 16)]
            cps.append(pltpu.async_copy(mn_hbm.at[gi_v[r]], rows[r], sems[r]))
        for r in range(R):
            for j in range(CH // 16):
                si[r][pl.ds(j * 16, 16)] = mo[r][pl.ds(level * CH + j * 16, 16)]
            cps[r].wait()
            pltpu.sync_copy(rows[r], accum.at[si[r]], add=True)
        return 0

    lax.fori_loop(0, M // R, group, 0)
    plsc.subcore_barrier()
    _copy_out_stripe(ts, accum, msg_hbm.at[c])


def _msg(level):
    return pl.kernel(
        functools.partial(_msg_kernel, level), mesh=_MESH,
        out_type=jax.ShapeDtypeStruct((2, N, D), jnp.float32),
        scratch_types=[
            [pltpu.VMEM((4 * CH,), jnp.int32) for _ in range(R)],
            [pltpu.VMEM((CH,), jnp.int32) for _ in range(R)],
            [pltpu.VMEM((CH,), jnp.int32) for _ in range(R)],
            [pltpu.VMEM((CH, D), jnp.float32) for _ in range(R)],
            pltpu.VMEM_SHARED((NACC, D), jnp.float32),
            [pltpu.SemaphoreType.DMA for _ in range(R)],
        ],
    )


def kernel(x, edge_index, gate, forward_level, forward_index, Ws, Wt, W_hs,
           b_hs, aggr_W, aggr_b, gru_Wih, gru_bih, gru_Whh, gru_bhh):
    f32 = jnp.float32
    src = jnp.pad(edge_index[0].astype(jnp.int32), (0, EP - E))
    dst = jnp.pad(edge_index[1].astype(jnp.int32), (0, EP - E))
    gate2 = gate.astype(jnp.int32).reshape(N, 1)
    lv2 = forward_level.astype(jnp.int32).reshape(N, 1)
    ws_p = jnp.pad(Ws.astype(f32), ((0, 2), (0, 0)))
    wt_p = jnp.pad(Wt.astype(f32), ((0, 2), (0, 0)))
    wa = W_hs[:D].astype(f32)
    wb = W_hs[D:].astype(f32)
    b2 = b_hs.reshape(1, D).astype(f32)
    at = aggr_W[:, :D, :].astype(f32)
    abm = aggr_W[:, D:, :].astype(f32)
    ab2 = aggr_b.reshape(4, 1, D).astype(f32)
    bih2 = gru_bih.reshape(4, 1, 3 * D).astype(f32)
    bhh2 = gru_bhh.reshape(4, 1, 3 * D).astype(f32)
    wih = gru_Wih.astype(f32)
    whh = gru_Whh.astype(f32)
    zrows = jnp.zeros((ZR, D), f32)

    xs, xt, pk = _encode(x.astype(jnp.int32), gate2, lv2, ws_p, wt_p)
    ags, agt, midx = _segsum(xs, xt, src, dst, pk.reshape(N), zrows)
    hs, mn = _mix(xs, xt, ags, agt, wa, wb, b2, at, ab2)
    hf = jnp.zeros((N, D), f32)
    for level in range(1, 4):
        msgp = _msg(level)(mn.reshape(4 * N, D), midx, zrows)
        outs = _gru(level, level < 3, msgp, hs, hf, gate2, lv2, wih, bih2,
                    whh, bhh2, at, abm, ab2)
        if level < 3:
            hf, mn = outs
        else:
            hf, = outs
    return hs, hf


# speedup vs baseline: 12.2259x; 1.0074x over previous
import functools

import jax
import jax.numpy as jnp
from jax import lax
from jax.experimental import pallas as pl
from jax.experimental.pallas import tpu as pltpu
from jax.experimental.pallas import tpu_sc as plsc

N = 10000
E = 160000
D = 128

NC = 2
NS = 16
NW = NC * NS
CH = 128
PT = 5120
EP = PT * NW
PTA = EP // NS
NACC = 10240
NST = 640
NSTL = N - 15 * NST
ZR = NACC // NS
NB = 1000
R = 2
CAPT = PT + 2 * CH
LW = 2 * CAPT


def _encode_body(x_ref, g_ref, l_ref, ws_ref, wt_ref, xs_ref, xt_ref, pk_ref):
    col = x_ref[:, 1:2]
    oh = (col == lax.broadcasted_iota(jnp.int32, (NB, 8), 1)).astype(jnp.float32)
    xs_ref[...] = jnp.dot(oh, ws_ref[...], preferred_element_type=jnp.float32)
    xt_ref[...] = jnp.dot(oh, wt_ref[...], preferred_element_type=jnp.float32)
    pk_ref[...] = g_ref[...] + 8 * l_ref[...]


def _encode(x, gate2, lv2, ws_p, wt_p):
    return pl.pallas_call(
        _encode_body,
        grid=(N // NB,),
        in_specs=[
            pl.BlockSpec((NB, 2), lambda i: (i, 0)),
            pl.BlockSpec((NB, 1), lambda i: (i, 0)),
            pl.BlockSpec((NB, 1), lambda i: (i, 0)),
            pl.BlockSpec((8, D), lambda i: (0, 0)),
            pl.BlockSpec((8, D), lambda i: (0, 0)),
        ],
        out_specs=[
            pl.BlockSpec((NB, D), lambda i: (i, 0)),
            pl.BlockSpec((NB, D), lambda i: (i, 0)),
            pl.BlockSpec((NB, 1), lambda i: (i, 0)),
        ],
        out_shape=[
            jax.ShapeDtypeStruct((N, D), jnp.float32),
            jax.ShapeDtypeStruct((N, D), jnp.float32),
            jax.ShapeDtypeStruct((N, 1), jnp.int32),
        ],
    )(x, gate2, lv2, ws_p, wt_p)


def _mix_body(xs_ref, xt_ref, ags_ref, agt_ref, wa_ref, wb_ref, b_ref,
              at_ref, ab_ref, hs_ref, mn_ref):
    s = jax.nn.relu(xs_ref[...] + ags_ref[...])
    t = jax.nn.relu(xt_ref[...] + agt_ref[...])
    hsb = (jnp.dot(s, wa_ref[...], preferred_element_type=jnp.float32)
           + jnp.dot(t, wb_ref[...], preferred_element_type=jnp.float32)
           + b_ref[...])
    hs_ref[...] = hsb
    for g in range(4):
        mn_ref[g] = jax.nn.relu(
            jnp.dot(hsb, at_ref[g], preferred_element_type=jnp.float32)
            + ab_ref[g])


def _mix(xs, xt, ags, agt, wa, wb, b2, at, ab2):
    return pl.pallas_call(
        _mix_body,
        grid=(N // NB,),
        in_specs=[
            pl.BlockSpec((NB, D), lambda i: (i, 0)),
            pl.BlockSpec((NB, D), lambda i: (i, 0)),
            pl.BlockSpec((NB, D), lambda i: (i, 0)),
            pl.BlockSpec((NB, D), lambda i: (i, 0)),
            pl.BlockSpec((D, D), lambda i: (0, 0)),
            pl.BlockSpec((D, D), lambda i: (0, 0)),
            pl.BlockSpec((1, D), lambda i: (0, 0)),
            pl.BlockSpec((4, D, D), lambda i: (0, 0, 0)),
            pl.BlockSpec((4, 1, D), lambda i: (0, 0, 0)),
        ],
        out_specs=[
            pl.BlockSpec((NB, D), lambda i: (i, 0)),
            pl.BlockSpec((4, NB, D), lambda i: (0, i, 0)),
        ],
        out_shape=[
            jax.ShapeDtypeStruct((N, D), jnp.float32),
            jax.ShapeDtypeStruct((4, N, D), jnp.float32),
        ],
    )(xs, xt, ags, agt, wa, wb, b2, at, ab2)


def _gru_body(level, with_mn, msgp_ref, hs_ref, hf_ref, gate_ref, lv_ref,
              wih_ref, bih_ref, whh_ref, bhh_ref, at_ref, abm_ref, ab_ref,
              hfo_ref, *mn_out):
    m = msgp_ref[0] + msgp_ref[1]
    g = gate_ref[...]
    gi = jnp.where(g == 2, 0, jnp.where(g == 3, 1,
         jnp.where(g == 4, 2, jnp.where(g == 1, 3, -1))))
    oh = (gi == lax.broadcasted_iota(jnp.int32, (NB, 4), 1)).astype(jnp.float32)
    hf = hf_ref[...]
    gi_lin = jnp.zeros((NB, 3 * D), jnp.float32)
    gh_lin = jnp.zeros((NB, 3 * D), jnp.float32)
    for gg in range(4):
        ohg = oh[:, gg:gg + 1]
        gi_lin = gi_lin + jnp.dot(m * ohg, wih_ref[gg],
                                  preferred_element_type=jnp.float32) + ohg * bih_ref[gg]
        gh_lin = gh_lin + jnp.dot(hf * ohg, whh_ref[gg],
                                  preferred_element_type=jnp.float32) + ohg * bhh_ref[gg]
    r = jax.nn.sigmoid(gi_lin[:, :D] + gh_lin[:, :D])
    z = jax.nn.sigmoid(gi_lin[:, D:2 * D] + gh_lin[:, D:2 * D])
    ng = jnp.tanh(gi_lin[:, 2 * D:] + r * gh_lin[:, 2 * D:])
    h_new = (1.0 - z) * ng + z * hf
    mask = (lv_ref[...] == level) & (gi >= 0)
    hf_new = jnp.where(mask, h_new, hf)
    hfo_ref[...] = hf_new
    if with_mn:
        hsb = hs_ref[...]
        for gg in range(4):
            mn_out[0][gg] = jax.nn.relu(
                jnp.dot(hsb, at_ref[gg], preferred_element_type=jnp.float32)
                + jnp.dot(hf_new, abm_ref[gg], preferred_element_type=jnp.float32)
                + ab_ref[gg])


def _gru(level, with_mn, msgp, hs, hf, gate2, lv2, wih, bih2, whh, bhh2,
         at, abm, ab2):
    out_specs = [pl.BlockSpec((NB, D), lambda i: (i, 0))]
    out_shape = [jax.ShapeDtypeStruct((N, D), jnp.float32)]
    if with_mn:
        out_specs.append(pl.BlockSpec((4, NB, D), lambda i: (0, i, 0)))
        out_shape.append(jax.ShapeDtypeStruct((4, N, D), jnp.float32))
    return pl.pallas_call(
        functools.partial(_gru_body, level, with_mn),
        grid=(N // NB,),
        in_specs=[
            pl.BlockSpec((2, NB, D), lambda i: (0, i, 0)),
            pl.BlockSpec((NB, D), lambda i: (i, 0)),
            pl.BlockSpec((NB, D), lambda i: (i, 0)),
            pl.BlockSpec((NB, 1), lambda i: (i, 0)),
            pl.BlockSpec((NB, 1), lambda i: (i, 0)),
            pl.BlockSpec((4, D, 3 * D), lambda i: (0, 0, 0)),
            pl.BlockSpec((4, 1, 3 * D), lambda i: (0, 0, 0)),
            pl.BlockSpec((4, D, 3 * D), lambda i: (0, 0, 0)),
            pl.BlockSpec((4, 1, 3 * D), lambda i: (0, 0, 0)),
            pl.BlockSpec((4, D, D), lambda i: (0, 0, 0)),
            pl.BlockSpec((4, D, D), lambda i: (0, 0, 0)),
            pl.BlockSpec((4, 1, D), lambda i: (0, 0, 0)),
        ],
        out_specs=out_specs,
        out_shape=out_shape,
    )(msgp, hs, hf, gate2, lv2, wih, bih2, whh, bhh2, at, abm, ab2)


_MESH = plsc.VectorSubcoreMesh(core_axis_name="c", subcore_axis_name="s")


def _copy_out_stripe(ts, accum, out_hbm):
    @pl.when(ts < 15)
    def _():
        off = pl.multiple_of(ts * NST, 8)
        pltpu.sync_copy(accum.at[pl.ds(off, NST)], out_hbm.at[pl.ds(off, NST)])

    @pl.when(ts == 15)
    def _():
        pltpu.sync_copy(accum.at[pl.ds(15 * NST, NSTL)],
                        out_hbm.at[pl.ds(15 * NST, NSTL)])


def _segsum_kernel(xs_hbm, xt_hbm, sd_hbm, pk_hbm, z_hbm,
                   aggs_hbm, aggt_hbm, midx_hbm,
                   sd, gb, si, rows, db, pv, mo_v, accum, sems, semm):
    c = lax.axis_index("c")
    ts = lax.axis_index("s")
    wid = ts * NC + c

    pltpu.sync_copy(z_hbm, accum.at[pl.ds(ts * ZR, ZR)])

    M1 = PT // CH

    def meta_chunk(ci, _):
        off = pl.multiple_of(wid * PT + ci * CH, CH)
        pltpu.sync_copy(sd_hbm.at[pl.ds(off * 2, 2 * CH)], sd[0])
        for j in range(CH // 16):
            db[pl.ds(j * 16, 16)] = sd[0][pl.ds(CH + j * 16, 16)]
        pltpu.async_copy(pk_hbm.at[db], pv, semm).wait()
        lane16 = lax.iota(jnp.int32, 16)
        for j in range(CH // 16):
            sl = pl.ds(j * 16, 16)
            eidx = off + j * 16 + lane16
            p = pv[sl]
            g = p & 7
            lv = jnp.where(eidx < E, p >> 3, -1)
            gi = jnp.where(g == 2, 0, jnp.where(g == 3, 1,
                 jnp.where(g == 4, 2, jnp.where(g == 1, 3, -1))))
            tr = N + (eidx & 15)
            d16 = db[sl]
            mo_v[sl] = jnp.maximum(gi, 0) * N + sd[0][sl]
            for l in range(1, 4):
                mo_v[pl.ds(l * CH + j * 16, 16)] = jnp.where(
                    (lv == l) & (gi >= 0), d16, tr)
        pltpu.sync_copy(mo_v, midx_hbm.at[pl.ds(off * 4, 4 * CH)])
        return 0

    lax.fori_loop(0, M1, meta_chunk, 0)
    plsc.subcore_barrier()

    M2 = PTA // CH

    def agg_group(tab_hbm, gslot, sslot, k):
        base = pl.multiple_of(ts * PTA + k * (R * CH), CH)
        cps = []
        for r in range(R):
            off = base + r * CH
            pltpu.sync_copy(sd_hbm.at[pl.ds(off * 2, 2 * CH)], sd[r])
            for j in range(CH // 16):
                gb[r][pl.ds(j * 16, 16)] = sd[r][pl.ds(gslot * CH + j * 16, 16)]
            cps.append(pltpu.async_copy(tab_hbm.at[gb[r]], rows[r], sems[r]))
        for r in range(R):
            off = base + r * CH
            for j in range(CH // 16):
                sl = pl.ds(j * 16, 16)
                eidx = off + j * 16 + lax.iota(jnp.int32, 16)
                raw = sd[r][pl.ds(sslot * CH + j * 16, 16)]
                si[r][sl] = jnp.where(eidx < E, raw, N + ts)
            cps[r].wait()
            pltpu.sync_copy(rows[r], accum.at[si[r]], add=True)
        return 0

    @pl.when(c == 0)
    def _():
        lax.fori_loop(0, M2 // R,
                      lambda k, _: agg_group(xs_hbm, 0, 1, k), 0)

    @pl.when(c == 1)
    def _():
        lax.fori_loop(0, M2 // R,
                      lambda k, _: agg_group(xt_hbm, 1, 0, k), 0)

    plsc.subcore_barrier()

    @pl.when(c == 0)
    def _():
        _copy_out_stripe(ts, accum, aggs_hbm)

    @pl.when(c == 1)
    def _():
        _copy_out_stripe(ts, accum, aggt_hbm)


_segsum = pl.kernel(
    _segsum_kernel, mesh=_MESH,
    out_type=[
        jax.ShapeDtypeStruct((N, D), jnp.float32),
        jax.ShapeDtypeStruct((N, D), jnp.float32),
        jax.ShapeDtypeStruct((4 * EP,), jnp.int32),
    ],
    scratch_types=[
        [pltpu.VMEM((2 * CH,), jnp.int32) for _ in range(R)],
        [pltpu.VMEM((CH,), jnp.int32) for _ in range(R)],
        [pltpu.VMEM((CH,), jnp.int32) for _ in range(R)],
        [pltpu.VMEM((CH, D), jnp.float32) for _ in range(R)],
        pltpu.VMEM((CH,), jnp.int32),
        pltpu.VMEM((CH,), jnp.int32),
        pltpu.VMEM((4 * CH,), jnp.int32),
        pltpu.VMEM_SHARED((NACC, D), jnp.float32),
        [pltpu.SemaphoreType.DMA for _ in range(R)],
        pltpu.SemaphoreType.DMA,
    ],
)


def _msg_kernel(level, mn_hbm, midx_hbm, z_hbm, msg_hbm,
                mo, gi_v, si, rows, accum, sems):
    c = lax.axis_index("c")
    ts = lax.axis_index("s")
    wid = ts * NC + c

    pltpu.sync_copy(z_hbm, accum.at[pl.ds(ts * ZR, ZR)])
    plsc.subcore_barrier()

    M = PT // CH

    def group(k, _):
        base = pl.multiple_of((wid * PT + k * (R * CH)) * 4, 4 * CH)
        cps = []
        for r in range(R):
            pltpu.sync_copy(midx_hbm.at[pl.ds(base + r * (4 * CH), 4 * CH)], mo[r])
            for j in range(CH // 16):
                gi_v[r][pl.ds(j * 16, 16)] = mo[r][pl.ds(j * 16, 16)]
            cps.append(pltpu.async_copy(mn_hbm.at[gi_v[r]], rows[r], sems[r]))
        for r in range(R):
            for j in range(CH // 16):
                si[r][pl.ds(j * 16, 16)] = mo[r][pl.ds(level * CH + j * 16, 16)]
            cps[r].wait()
            pltpu.sync_copy(rows[r], accum.at[si[r]], add=True)
        return 0

    lax.fori_loop(0, M // R, group, 0)
    plsc.subcore_barrier()
    _copy_out_stripe(ts, accum, msg_hbm.at[c])


def _msg(level):
    return pl.kernel(
        functools.partial(_msg_kernel, level), mesh=_MESH,
        out_type=jax.ShapeDtypeStruct((2, N, D), jnp.float32),
        scratch_types=[
            [pltpu.VMEM((4 * CH,), jnp.int32) for _ in range(R)],
            [pltpu.VMEM((CH,), jnp.int32) for _ in range(R)],
            [pltpu.VMEM((CH,), jnp.int32) for _ in range(R)],
            [pltpu.VMEM((CH, D), jnp.float32) for _ in range(R)],
            pltpu.VMEM_SHARED((NACC, D), jnp.float32),
            [pltpu.SemaphoreType.DMA for _ in range(R)],
        ],
    )


def kernel(x, edge_index, gate, forward_level, forward_index, Ws, Wt, W_hs,
           b_hs, aggr_W, aggr_b, gru_Wih, gru_bih, gru_Whh, gru_bhh):
    f32 = jnp.float32
    src = jnp.pad(edge_index[0].astype(jnp.int32), (0, EP - E))
    dst = jnp.pad(edge_index[1].astype(jnp.int32), (0, EP - E))
    sdpack = jnp.stack([src.reshape(EP // CH, CH),
                        dst.reshape(EP // CH, CH)], axis=1).reshape(2 * EP)
    gate2 = gate.astype(jnp.int32).reshape(N, 1)
    lv2 = forward_level.astype(jnp.int32).reshape(N, 1)
    ws_p = jnp.pad(Ws.astype(f32), ((0, 2), (0, 0)))
    wt_p = jnp.pad(Wt.astype(f32), ((0, 2), (0, 0)))
    wa = W_hs[:D].astype(f32)
    wb = W_hs[D:].astype(f32)
    b2 = b_hs.reshape(1, D).astype(f32)
    at = aggr_W[:, :D, :].astype(f32)
    abm = aggr_W[:, D:, :].astype(f32)
    ab2 = aggr_b.reshape(4, 1, D).astype(f32)
    bih2 = gru_bih.reshape(4, 1, 3 * D).astype(f32)
    bhh2 = gru_bhh.reshape(4, 1, 3 * D).astype(f32)
    wih = gru_Wih.astype(f32)
    whh = gru_Whh.astype(f32)
    zrows = jnp.zeros((ZR, D), f32)

    xs, xt, pk = _encode(x.astype(jnp.int32), gate2, lv2, ws_p, wt_p)
    ags, agt, midx = _segsum(xs, xt, sdpack, pk.reshape(N), zrows)
    hs, mn = _mix(xs, xt, ags, agt, wa, wb, b2, at, ab2)
    hf = jnp.zeros((N, D), f32)
    for level in range(1, 4):
        msgp = _msg(level)(mn.reshape(4 * N, D), midx, zrows)
        outs = _gru(level, level < 3, msgp, hs, hf, gate2, lv2, wih, bih2,
                    whh, bhh2, at, abm, ab2)
        if level < 3:
            hf, mn = outs
        else:
            hf, = outs
    return hs, hf


# speedup vs baseline: 12.2356x; 1.0008x over previous
import functools

import jax
import jax.numpy as jnp
from jax import lax
from jax.experimental import pallas as pl
from jax.experimental.pallas import tpu as pltpu
from jax.experimental.pallas import tpu_sc as plsc

N = 10000
E = 160000
D = 128

NC = 2
NS = 16
NW = NC * NS
CH = 128
PT = 5120
EP = PT * NW
PTA = EP // NS
NACC = 10240
NST = 640
NSTL = N - 15 * NST
ZR = NACC // NS
NB = 1000
R = 2
CAPT = PT + 2 * CH
LW = 2 * CAPT


def _encode_body(x_ref, g_ref, l_ref, ws_ref, wt_ref, xs_ref, xt_ref, pk_ref):
    col = x_ref[:, 1:2]
    oh = (col == lax.broadcasted_iota(jnp.int32, (NB, 8), 1)).astype(jnp.float32)
    xs_ref[...] = jnp.dot(oh, ws_ref[...], preferred_element_type=jnp.float32)
    xt_ref[...] = jnp.dot(oh, wt_ref[...], preferred_element_type=jnp.float32)
    pk_ref[...] = g_ref[...] + 8 * l_ref[...]


def _encode(x, gate2, lv2, ws_p, wt_p):
    return pl.pallas_call(
        _encode_body,
        grid=(N // NB,),
        in_specs=[
            pl.BlockSpec((NB, 2), lambda i: (i, 0)),
            pl.BlockSpec((NB, 1), lambda i: (i, 0)),
            pl.BlockSpec((NB, 1), lambda i: (i, 0)),
            pl.BlockSpec((8, D), lambda i: (0, 0)),
            pl.BlockSpec((8, D), lambda i: (0, 0)),
        ],
        out_specs=[
            pl.BlockSpec((NB, D), lambda i: (i, 0)),
            pl.BlockSpec((NB, D), lambda i: (i, 0)),
            pl.BlockSpec((NB, 1), lambda i: (i, 0)),
        ],
        out_shape=[
            jax.ShapeDtypeStruct((N, D), jnp.float32),
            jax.ShapeDtypeStruct((N, D), jnp.float32),
            jax.ShapeDtypeStruct((N, 1), jnp.int32),
        ],
    )(x, gate2, lv2, ws_p, wt_p)


def _mix_body(xs_ref, xt_ref, ags_ref, agt_ref, wa_ref, wb_ref, b_ref,
              at_ref, ab_ref, hs_ref, mn_ref):
    s = jax.nn.relu(xs_ref[...] + ags_ref[...])
    t = jax.nn.relu(xt_ref[...] + agt_ref[...])
    hsb = (jnp.dot(s, wa_ref[...], preferred_element_type=jnp.float32)
           + jnp.dot(t, wb_ref[...], preferred_element_type=jnp.float32)
           + b_ref[...])
    hs_ref[...] = hsb
    for g in range(4):
        mn_ref[g] = jax.nn.relu(
            jnp.dot(hsb, at_ref[g], preferred_element_type=jnp.float32)
            + ab_ref[g])


def _mix(xs, xt, ags, agt, wa, wb, b2, at, ab2):
    return pl.pallas_call(
        _mix_body,
        grid=(N // NB,),
        in_specs=[
            pl.BlockSpec((NB, D), lambda i: (i, 0)),
            pl.BlockSpec((NB, D), lambda i: (i, 0)),
            pl.BlockSpec((NB, D), lambda i: (i, 0)),
            pl.BlockSpec((NB, D), lambda i: (i, 0)),
            pl.BlockSpec((D, D), lambda i: (0, 0)),
            pl.BlockSpec((D, D), lambda i: (0, 0)),
            pl.BlockSpec((1, D), lambda i: (0, 0)),
            pl.BlockSpec((4, D, D), lambda i: (0, 0, 0)),
            pl.BlockSpec((4, 1, D), lambda i: (0, 0, 0)),
        ],
        out_specs=[
            pl.BlockSpec((NB, D), lambda i: (i, 0)),
            pl.BlockSpec((4, NB, D), lambda i: (0, i, 0)),
        ],
        out_shape=[
            jax.ShapeDtypeStruct((N, D), jnp.float32),
            jax.ShapeDtypeStruct((4, N, D), jnp.float32),
        ],
    )(xs, xt, ags, agt, wa, wb, b2, at, ab2)


def _gru_body(level, with_mn, msgp_ref, hs_ref, hf_ref, gate_ref, lv_ref,
              wih_ref, bih_ref, whh_ref, bhh_ref, at_ref, abm_ref, ab_ref,
              hfo_ref, *mn_out):
    m = msgp_ref[0] + msgp_ref[1]
    g = gate_ref[...]
    gi = jnp.where(g == 2, 0, jnp.where(g == 3, 1,
         jnp.where(g == 4, 2, jnp.where(g == 1, 3, -1))))
    oh = (gi == lax.broadcasted_iota(jnp.int32, (NB, 4), 1)).astype(jnp.float32)
    hf = hf_ref[...]
    gi_lin = jnp.zeros((NB, 3 * D), jnp.float32)
    gh_lin = jnp.zeros((NB, 3 * D), jnp.float32)
    for gg in range(4):
        ohg = oh[:, gg:gg + 1]
        gi_lin = gi_lin + jnp.dot(m * ohg, wih_ref[gg],
                                  preferred_element_type=jnp.float32) + ohg * bih_ref[gg]
        gh_lin = gh_lin + jnp.dot(hf * ohg, whh_ref[gg],
                                  preferred_element_type=jnp.float32) + ohg * bhh_ref[gg]
    r = jax.nn.sigmoid(gi_lin[:, :D] + gh_lin[:, :D])
    z = jax.nn.sigmoid(gi_lin[:, D:2 * D] + gh_lin[:, D:2 * D])
    ng = jnp.tanh(gi_lin[:, 2 * D:] + r * gh_lin[:, 2 * D:])
    h_new = (1.0 - z) * ng + z * hf
    mask = (lv_ref[...] == level) & (gi >= 0)
    hf_new = jnp.where(mask, h_new, hf)
    hfo_ref[...] = hf_new
    if with_mn:
        hsb = hs_ref[...]
        for gg in range(4):
            mn_out[0][gg] = jax.nn.relu(
                jnp.dot(hsb, at_ref[gg], preferred_element_type=jnp.float32)
                + jnp.dot(hf_new, abm_ref[gg], preferred_element_type=jnp.float32)
                + ab_ref[gg])


def _gru(level, with_mn, msgp, hs, hf, gate2, lv2, wih, bih2, whh, bhh2,
         at, abm, ab2):
    out_specs = [pl.BlockSpec((NB, D), lambda i: (i, 0))]
    out_shape = [jax.ShapeDtypeStruct((N, D), jnp.float32)]
    if with_mn:
        out_specs.append(pl.BlockSpec((4, NB, D), lambda i: (0, i, 0)))
        out_shape.append(jax.ShapeDtypeStruct((4, N, D), jnp.float32))
    return pl.pallas_call(
        functools.partial(_gru_body, level, with_mn),
        grid=(N // NB,),
        in_specs=[
            pl.BlockSpec((2, NB, D), lambda i: (0, i, 0)),
            pl.BlockSpec((NB, D), lambda i: (i, 0)),
            pl.BlockSpec((NB, D), lambda i: (i, 0)),
            pl.BlockSpec((NB, 1), lambda i: (i, 0)),
            pl.BlockSpec((NB, 1), lambda i: (i, 0)),
            pl.BlockSpec((4, D, 3 * D), lambda i: (0, 0, 0)),
            pl.BlockSpec((4, 1, 3 * D), lambda i: (0, 0, 0)),
            pl.BlockSpec((4, D, 3 * D), lambda i: (0, 0, 0)),
            pl.BlockSpec((4, 1, 3 * D), lambda i: (0, 0, 0)),
            pl.BlockSpec((4, D, D), lambda i: (0, 0, 0)),
            pl.BlockSpec((4, D, D), lambda i: (0, 0, 0)),
            pl.BlockSpec((4, 1, D), lambda i: (0, 0, 0)),
        ],
        out_specs=out_specs,
        out_shape=out_shape,
    )(msgp, hs, hf, gate2, lv2, wih, bih2, whh, bhh2, at, abm, ab2)


_MESH = plsc.VectorSubcoreMesh(core_axis_name="c", subcore_axis_name="s")


def _copy_out_stripe(ts, accum, out_hbm):
    @pl.when(ts < 15)
    def _():
        off = pl.multiple_of(ts * NST, 8)
        pltpu.sync_copy(accum.at[pl.ds(off, NST)], out_hbm.at[pl.ds(off, NST)])

    @pl.when(ts == 15)
    def _():
        pltpu.sync_copy(accum.at[pl.ds(15 * NST, NSTL)],
                        out_hbm.at[pl.ds(15 * NST, NSTL)])


def _segsum_kernel(xs_hbm, xt_hbm, sd_hbm, pk_hbm, z_hbm,
                   aggs_hbm, aggt_hbm, midx_hbm,
                   sd, gb, si, rows, db, pv, mo_v, accum, sems, sems2, semm):
    c = lax.axis_index("c")
    ts = lax.axis_index("s")
    wid = ts * NC + c

    pltpu.sync_copy(z_hbm, accum.at[pl.ds(ts * ZR, ZR)])

    M1 = PT // CH

    def meta_chunk(ci, _):
        off = pl.multiple_of(wid * PT + ci * CH, CH)
        pltpu.sync_copy(sd_hbm.at[pl.ds(off * 2, 2 * CH)], sd[0])
        for j in range(CH // 16):
            db[pl.ds(j * 16, 16)] = sd[0][pl.ds(CH + j * 16, 16)]
        pltpu.async_copy(pk_hbm.at[db], pv, semm).wait()
        lane16 = lax.iota(jnp.int32, 16)
        for j in range(CH // 16):
            sl = pl.ds(j * 16, 16)
            eidx = off + j * 16 + lane16
            p = pv[sl]
            g = p & 7
            lv = jnp.where(eidx < E, p >> 3, -1)
            gi = jnp.where(g == 2, 0, jnp.where(g == 3, 1,
                 jnp.where(g == 4, 2, jnp.where(g == 1, 3, -1))))
            tr = N + (eidx & 15)
            d16 = db[sl]
            mo_v[sl] = jnp.maximum(gi, 0) * N + sd[0][sl]
            for l in range(1, 4):
                mo_v[pl.ds(l * CH + j * 16, 16)] = jnp.where(
                    (lv == l) & (gi >= 0), d16, tr)
        pltpu.sync_copy(mo_v, midx_hbm.at[pl.ds(off * 4, 4 * CH)])
        return 0

    lax.fori_loop(0, M1, meta_chunk, 0)
    plsc.subcore_barrier()

    M2 = PTA // CH

    def agg_group(tab_hbm, gslot, sslot, k):
        base = pl.multiple_of(ts * PTA + k * (R * CH), CH)
        cps = []
        scps = []
        for r in range(R):
            off = base + r * CH
            pltpu.sync_copy(sd_hbm.at[pl.ds(off * 2, 2 * CH)], sd[r])
            for j in range(CH // 16):
                gb[r][pl.ds(j * 16, 16)] = sd[r][pl.ds(gslot * CH + j * 16, 16)]
            cps.append(pltpu.async_copy(tab_hbm.at[gb[r]], rows[r], sems[r]))
        for r in range(R):
            off = base + r * CH
            for j in range(CH // 16):
                sl = pl.ds(j * 16, 16)
                eidx = off + j * 16 + lax.iota(jnp.int32, 16)
                raw = sd[r][pl.ds(sslot * CH + j * 16, 16)]
                si[r][sl] = jnp.where(eidx < E, raw, N + ts)
            cps[r].wait()
            scps.append(pltpu.async_copy(rows[r], accum.at[si[r]], sems2[r],
                                         add=True))
        for sc in scps:
            sc.wait()
        return 0

    @pl.when(c == 0)
    def _():
        lax.fori_loop(0, M2 // R,
                      lambda k, _: agg_group(xs_hbm, 0, 1, k), 0)

    @pl.when(c == 1)
    def _():
        lax.fori_loop(0, M2 // R,
                      lambda k, _: agg_group(xt_hbm, 1, 0, k), 0)

    plsc.subcore_barrier()

    @pl.when(c == 0)
    def _():
        _copy_out_stripe(ts, accum, aggs_hbm)

    @pl.when(c == 1)
    def _():
        _copy_out_stripe(ts, accum, aggt_hbm)


_segsum = pl.kernel(
    _segsum_kernel, mesh=_MESH,
    out_type=[
        jax.ShapeDtypeStruct((N, D), jnp.float32),
        jax.ShapeDtypeStruct((N, D), jnp.float32),
        jax.ShapeDtypeStruct((4 * EP,), jnp.int32),
    ],
    scratch_types=[
        [pltpu.VMEM((2 * CH,), jnp.int32) for _ in range(R)],
        [pltpu.VMEM((CH,), jnp.int32) for _ in range(R)],
        [pltpu.VMEM((CH,), jnp.int32) for _ in range(R)],
        [pltpu.VMEM((CH, D), jnp.float32) for _ in range(R)],
        pltpu.VMEM((CH,), jnp.int32),
        pltpu.VMEM((CH,), jnp.int32),
        pltpu.VMEM((4 * CH,), jnp.int32),
        pltpu.VMEM_SHARED((NACC, D), jnp.float32),
        [pltpu.SemaphoreType.DMA for _ in range(R)],
        [pltpu.SemaphoreType.DMA for _ in range(R)],
        pltpu.SemaphoreType.DMA,
    ],
)


def _msg_kernel(level, mn_hbm, midx_hbm, z_hbm, msg_hbm,
                mo, gi_v, si, rows, accum, sems, sems2):
    c = lax.axis_index("c")
    ts = lax.axis_index("s")
    wid = ts * NC + c

    pltpu.sync_copy(z_hbm, accum.at[pl.ds(ts * ZR, ZR)])
    plsc.subcore_barrier()

    M = PT // CH

    def group(k, _):
        base = pl.multiple_of((wid * PT + k * (R * CH)) * 4, 4 * CH)
        cps = []
        for r in range(R):
            pltpu.sync_copy(midx_hbm.at[pl.ds(base + r * (4 * CH), 4 * CH)], mo[r])
            for j in range(CH // 16):
                gi_v[r][pl.ds(j * 16, 16)] = mo[r][pl.ds(j * 16, 16)]
            cps.append(pltpu.async_copy(mn_hbm.at[gi_v[r]], rows[r], sems[r]))
        scps = []
        for r in range(R):
            for j in range(CH // 16):
                si[r][pl.ds(j * 16, 16)] = mo[r][pl.ds(level * CH + j * 16, 16)]
            cps[r].wait()
            scps.append(pltpu.async_copy(rows[r], accum.at[si[r]], sems2[r],
                                         add=True))
        for sc in scps:
            sc.wait()
        return 0

    lax.fori_loop(0, M // R, group, 0)
    plsc.subcore_barrier()
    _copy_out_stripe(ts, accum, msg_hbm.at[c])


def _msg(level):
    return pl.kernel(
        functools.partial(_msg_kernel, level), mesh=_MESH,
        out_type=jax.ShapeDtypeStruct((2, N, D), jnp.float32),
        scratch_types=[
            [pltpu.VMEM((4 * CH,), jnp.int32) for _ in range(R)],
            [pltpu.VMEM((CH,), jnp.int32) for _ in range(R)],
            [pltpu.VMEM((CH,), jnp.int32) for _ in range(R)],
            [pltpu.VMEM((CH, D), jnp.float32) for _ in range(R)],
            pltpu.VMEM_SHARED((NACC, D), jnp.float32),
            [pltpu.SemaphoreType.DMA for _ in range(R)],
            [pltpu.SemaphoreType.DMA for _ in range(R)],
        ],
    )


def kernel(x, edge_index, gate, forward_level, forward_index, Ws, Wt, W_hs,
           b_hs, aggr_W, aggr_b, gru_Wih, gru_bih, gru_Whh, gru_bhh):
    f32 = jnp.float32
    src = jnp.pad(edge_index[0].astype(jnp.int32), (0, EP - E))
    dst = jnp.pad(edge_index[1].astype(jnp.int32), (0, EP - E))
    sdpack = jnp.stack([src.reshape(EP // CH, CH),
                        dst.reshape(EP // CH, CH)], axis=1).reshape(2 * EP)
    gate2 = gate.astype(jnp.int32).reshape(N, 1)
    lv2 = forward_level.astype(jnp.int32).reshape(N, 1)
    ws_p = jnp.pad(Ws.astype(f32), ((0, 2), (0, 0)))
    wt_p = jnp.pad(Wt.astype(f32), ((0, 2), (0, 0)))
    wa = W_hs[:D].astype(f32)
    wb = W_hs[D:].astype(f32)
    b2 = b_hs.reshape(1, D).astype(f32)
    at = aggr_W[:, :D, :].astype(f32)
    abm = aggr_W[:, D:, :].astype(f32)
    ab2 = aggr_b.reshape(4, 1, D).astype(f32)
    bih2 = gru_bih.reshape(4, 1, 3 * D).astype(f32)
    bhh2 = gru_bhh.reshape(4, 1, 3 * D).astype(f32)
    wih = gru_Wih.astype(f32)
    whh = gru_Whh.astype(f32)
    zrows = jnp.zeros((ZR, D), f32)

    xs, xt, pk = _encode(x.astype(jnp.int32), gate2, lv2, ws_p, wt_p)
    ags, agt, midx = _segsum(xs, xt, sdpack, pk.reshape(N), zrows)
    hs, mn = _mix(xs, xt, ags, agt, wa, wb, b2, at, ab2)
    hf = jnp.zeros((N, D), f32)
    for level in range(1, 4):
        msgp = _msg(level)(mn.reshape(4 * N, D), midx, zrows)
        outs = _gru(level, level < 3, msgp, hs, hf, gate2, lv2, wih, bih2,
                    whh, bhh2, at, abm, ab2)
        if level < 3:
            hf, mn = outs
        else:
            hf, = outs
    return hs, hf


# speedup vs baseline: 12.2377x; 1.0002x over previous
import functools

import jax
import jax.numpy as jnp
from jax import lax
from jax.experimental import pallas as pl
from jax.experimental.pallas import tpu as pltpu
from jax.experimental.pallas import tpu_sc as plsc

N = 10000
E = 160000
D = 128

NC = 2
NS = 16
NW = NC * NS
CH = 128
PT = 5120
EP = PT * NW
PTA = EP // NS
NACC = 10240
NST = 640
NSTL = N - 15 * NST
ZR = NACC // NS
NB = 1000
R = 2
CAPT = PT + 2 * CH
LW = 2 * CAPT


def _encode_body(x_ref, g_ref, l_ref, ws_ref, wt_ref, xs_ref, xt_ref, pk_ref):
    col = x_ref[:, 1:2]
    oh = (col == lax.broadcasted_iota(jnp.int32, (NB, 8), 1)).astype(jnp.float32)
    xs_ref[...] = jnp.dot(oh, ws_ref[...], preferred_element_type=jnp.float32)
    xt_ref[...] = jnp.dot(oh, wt_ref[...], preferred_element_type=jnp.float32)
    pk_ref[...] = g_ref[...] + 8 * l_ref[...]


def _encode(x, gate2, lv2, ws_p, wt_p):
    return pl.pallas_call(
        _encode_body,
        grid=(N // NB,),
        in_specs=[
            pl.BlockSpec((NB, 2), lambda i: (i, 0)),
            pl.BlockSpec((NB, 1), lambda i: (i, 0)),
            pl.BlockSpec((NB, 1), lambda i: (i, 0)),
            pl.BlockSpec((8, D), lambda i: (0, 0)),
            pl.BlockSpec((8, D), lambda i: (0, 0)),
        ],
        out_specs=[
            pl.BlockSpec((NB, D), lambda i: (i, 0)),
            pl.BlockSpec((NB, D), lambda i: (i, 0)),
            pl.BlockSpec((NB, 1), lambda i: (i, 0)),
        ],
        out_shape=[
            jax.ShapeDtypeStruct((N, D), jnp.float32),
            jax.ShapeDtypeStruct((N, D), jnp.float32),
            jax.ShapeDtypeStruct((N, 1), jnp.int32),
        ],
    )(x, gate2, lv2, ws_p, wt_p)


def _mix_body(xs_ref, xt_ref, ags_ref, agt_ref, wa_ref, wb_ref, b_ref,
              at_ref, ab_ref, hs_ref, mn_ref):
    s = jax.nn.relu(xs_ref[...] + ags_ref[...])
    t = jax.nn.relu(xt_ref[...] + agt_ref[...])
    hsb = (jnp.dot(s, wa_ref[...], preferred_element_type=jnp.float32)
           + jnp.dot(t, wb_ref[...], preferred_element_type=jnp.float32)
           + b_ref[...])
    hs_ref[...] = hsb
    for g in range(4):
        mn_ref[g] = jax.nn.relu(
            jnp.dot(hsb, at_ref[g], preferred_element_type=jnp.float32)
            + ab_ref[g])


def _mix(xs, xt, ags, agt, wa, wb, b2, at, ab2):
    return pl.pallas_call(
        _mix_body,
        grid=(N // NB,),
        in_specs=[
            pl.BlockSpec((NB, D), lambda i: (i, 0)),
            pl.BlockSpec((NB, D), lambda i: (i, 0)),
            pl.BlockSpec((NB, D), lambda i: (i, 0)),
            pl.BlockSpec((NB, D), lambda i: (i, 0)),
            pl.BlockSpec((D, D), lambda i: (0, 0)),
            pl.BlockSpec((D, D), lambda i: (0, 0)),
            pl.BlockSpec((1, D), lambda i: (0, 0)),
            pl.BlockSpec((4, D, D), lambda i: (0, 0, 0)),
            pl.BlockSpec((4, 1, D), lambda i: (0, 0, 0)),
        ],
        out_specs=[
            pl.BlockSpec((NB, D), lambda i: (i, 0)),
            pl.BlockSpec((4, NB, D), lambda i: (0, i, 0)),
        ],
        out_shape=[
            jax.ShapeDtypeStruct((N, D), jnp.float32),
            jax.ShapeDtypeStruct((4, N, D), jnp.float32),
        ],
    )(xs, xt, ags, agt, wa, wb, b2, at, ab2)


def _gru_body(level, with_mn, msgp_ref, hs_ref, hf_ref, gate_ref, lv_ref,
              wih_ref, bih_ref, whh_ref, bhh_ref, at_ref, abm_ref, ab_ref,
              hfo_ref, *mn_out):
    m = msgp_ref[0] + msgp_ref[1]
    g = gate_ref[...]
    gi = jnp.where(g == 2, 0, jnp.where(g == 3, 1,
         jnp.where(g == 4, 2, jnp.where(g == 1, 3, -1))))
    oh = (gi == lax.broadcasted_iota(jnp.int32, (NB, 4), 1)).astype(jnp.float32)
    hf = hf_ref[...]
    gi_lin = jnp.zeros((NB, 3 * D), jnp.float32)
    gh_lin = jnp.zeros((NB, 3 * D), jnp.float32)
    for gg in range(4):
        ohg = oh[:, gg:gg + 1]
        gi_lin = gi_lin + jnp.dot(m * ohg, wih_ref[gg],
                                  preferred_element_type=jnp.float32) + ohg * bih_ref[gg]
        gh_lin = gh_lin + jnp.dot(hf * ohg, whh_ref[gg],
                                  preferred_element_type=jnp.float32) + ohg * bhh_ref[gg]
    r = jax.nn.sigmoid(gi_lin[:, :D] + gh_lin[:, :D])
    z = jax.nn.sigmoid(gi_lin[:, D:2 * D] + gh_lin[:, D:2 * D])
    ng = jnp.tanh(gi_lin[:, 2 * D:] + r * gh_lin[:, 2 * D:])
    h_new = (1.0 - z) * ng + z * hf
    mask = (lv_ref[...] == level) & (gi >= 0)
    hf_new = jnp.where(mask, h_new, hf)
    hfo_ref[...] = hf_new
    if with_mn:
        hsb = hs_ref[...]
        for gg in range(4):
            mn_out[0][gg] = jax.nn.relu(
                jnp.dot(hsb, at_ref[gg], preferred_element_type=jnp.float32)
                + jnp.dot(hf_new, abm_ref[gg], preferred_element_type=jnp.float32)
                + ab_ref[gg])


def _gru(level, with_mn, msgp, hs, hf, gate2, lv2, wih, bih2, whh, bhh2,
         at, abm, ab2):
    out_specs = [pl.BlockSpec((NB, D), lambda i: (i, 0))]
    out_shape = [jax.ShapeDtypeStruct((N, D), jnp.float32)]
    if with_mn:
        out_specs.append(pl.BlockSpec((4, NB, D), lambda i: (0, i, 0)))
        out_shape.append(jax.ShapeDtypeStruct((4, N, D), jnp.float32))
    return pl.pallas_call(
        functools.partial(_gru_body, level, with_mn),
        grid=(N // NB,),
        in_specs=[
            pl.BlockSpec((2, NB, D), lambda i: (0, i, 0)),
            pl.BlockSpec((NB, D), lambda i: (i, 0)),
            pl.BlockSpec((NB, D), lambda i: (i, 0)),
            pl.BlockSpec((NB, 1), lambda i: (i, 0)),
            pl.BlockSpec((NB, 1), lambda i: (i, 0)),
            pl.BlockSpec((4, D, 3 * D), lambda i: (0, 0, 0)),
            pl.BlockSpec((4, 1, 3 * D), lambda i: (0, 0, 0)),
            pl.BlockSpec((4, D, 3 * D), lambda i: (0, 0, 0)),
            pl.BlockSpec((4, 1, 3 * D), lambda i: (0, 0, 0)),
            pl.BlockSpec((4, D, D), lambda i: (0, 0, 0)),
            pl.BlockSpec((4, D, D), lambda i: (0, 0, 0)),
            pl.BlockSpec((4, 1, D), lambda i: (0, 0, 0)),
        ],
        out_specs=out_specs,
        out_shape=out_shape,
    )(msgp, hs, hf, gate2, lv2, wih, bih2, whh, bhh2, at, abm, ab2)


_MESH = plsc.VectorSubcoreMesh(core_axis_name="c", subcore_axis_name="s")


def _copy_out_stripe(ts, accum, out_hbm):
    @pl.when(ts < 15)
    def _():
        off = pl.multiple_of(ts * NST, 8)
        pltpu.sync_copy(accum.at[pl.ds(off, NST)], out_hbm.at[pl.ds(off, NST)])

    @pl.when(ts == 15)
    def _():
        pltpu.sync_copy(accum.at[pl.ds(15 * NST, NSTL)],
                        out_hbm.at[pl.ds(15 * NST, NSTL)])


def _segsum_kernel(xs_hbm, xt_hbm, sd_hbm, pk_hbm, z_hbm,
                   aggs_hbm, aggt_hbm, midx_hbm,
                   sd, gb, si, rows, db, pv, mo_v, accum, sems, sems2, semm):
    c = lax.axis_index("c")
    ts = lax.axis_index("s")
    wid = ts * NC + c

    pltpu.sync_copy(z_hbm, accum.at[pl.ds(ts * ZR, ZR)])

    M1 = PT // CH

    def meta_chunk(ci, _):
        off = pl.multiple_of(wid * PT + ci * CH, CH)
        pltpu.sync_copy(sd_hbm.at[pl.ds(off * 2, 2 * CH)], sd[0])
        for j in range(CH // 16):
            db[pl.ds(j * 16, 16)] = sd[0][pl.ds(CH + j * 16, 16)]
        pltpu.async_copy(pk_hbm.at[db], pv, semm).wait()
        lane16 = lax.iota(jnp.int32, 16)
        for j in range(CH // 16):
            sl = pl.ds(j * 16, 16)
            eidx = off + j * 16 + lane16
            p = pv[sl]
            g = p & 7
            lv = jnp.where(eidx < E, p >> 3, -1)
            gi = jnp.where(g == 2, 0, jnp.where(g == 3, 1,
                 jnp.where(g == 4, 2, jnp.where(g == 1, 3, -1))))
            tr = N + (eidx & 127)
            d16 = db[sl]
            mo_v[sl] = jnp.maximum(gi, 0) * N + sd[0][sl]
            for l in range(1, 4):
                mo_v[pl.ds(l * CH + j * 16, 16)] = jnp.where(
                    (lv == l) & (gi >= 0), d16, tr)
        pltpu.sync_copy(mo_v, midx_hbm.at[pl.ds(off * 4, 4 * CH)])
        return 0

    lax.fori_loop(0, M1, meta_chunk, 0)
    plsc.subcore_barrier()

    M2 = PTA // CH

    def agg_group(tab_hbm, gslot, sslot, k):
        base = pl.multiple_of(ts * PTA + k * (R * CH), CH)
        cps = []
        scps = []
        for r in range(R):
            off = base + r * CH
            pltpu.sync_copy(sd_hbm.at[pl.ds(off * 2, 2 * CH)], sd[r])
            for j in range(CH // 16):
                gb[r][pl.ds(j * 16, 16)] = sd[r][pl.ds(gslot * CH + j * 16, 16)]
            cps.append(pltpu.async_copy(tab_hbm.at[gb[r]], rows[r], sems[r]))
        for r in range(R):
            off = base + r * CH
            for j in range(CH // 16):
                sl = pl.ds(j * 16, 16)
                eidx = off + j * 16 + lax.iota(jnp.int32, 16)
                raw = sd[r][pl.ds(sslot * CH + j * 16, 16)]
                si[r][sl] = jnp.where(eidx < E, raw, N + ts)
            cps[r].wait()
            scps.append(pltpu.async_copy(rows[r], accum.at[si[r]], sems2[r],
                                         add=True))
        for sc in scps:
            sc.wait()
        return 0

    @pl.when(c == 0)
    def _():
        lax.fori_loop(0, M2 // R,
                      lambda k, _: agg_group(xs_hbm, 0, 1, k), 0)

    @pl.when(c == 1)
    def _():
        lax.fori_loop(0, M2 // R,
                      lambda k, _: agg_group(xt_hbm, 1, 0, k), 0)

    plsc.subcore_barrier()

    @pl.when(c == 0)
    def _():
        _copy_out_stripe(ts, accum, aggs_hbm)

    @pl.when(c == 1)
    def _():
        _copy_out_stripe(ts, accum, aggt_hbm)


_segsum = pl.kernel(
    _segsum_kernel, mesh=_MESH,
    out_type=[
        jax.ShapeDtypeStruct((N, D), jnp.float32),
        jax.ShapeDtypeStruct((N, D), jnp.float32),
        jax.ShapeDtypeStruct((4 * EP,), jnp.int32),
    ],
    scratch_types=[
        [pltpu.VMEM((2 * CH,), jnp.int32) for _ in range(R)],
        [pltpu.VMEM((CH,), jnp.int32) for _ in range(R)],
        [pltpu.VMEM((CH,), jnp.int32) for _ in range(R)],
        [pltpu.VMEM((CH, D), jnp.float32) for _ in range(R)],
        pltpu.VMEM((CH,), jnp.int32),
        pltpu.VMEM((CH,), jnp.int32),
        pltpu.VMEM((4 * CH,), jnp.int32),
        pltpu.VMEM_SHARED((NACC, D), jnp.float32),
        [pltpu.SemaphoreType.DMA for _ in range(R)],
        [pltpu.SemaphoreType.DMA for _ in range(R)],
        pltpu.SemaphoreType.DMA,
    ],
)


def _msg_kernel(level, mn_hbm, midx_hbm, z_hbm, msg_hbm,
                mo, gi_v, si, rows, accum, sems, sems2):
    c = lax.axis_index("c")
    ts = lax.axis_index("s")
    wid = ts * NC + c

    pltpu.sync_copy(z_hbm, accum.at[pl.ds(ts * ZR, ZR)])
    plsc.subcore_barrier()

    M = PT // CH

    def group(k, _):
        base = pl.multiple_of((wid * PT + k * (R * CH)) * 4, 4 * CH)
        cps = []
        for r in range(R):
            pltpu.sync_copy(midx_hbm.at[pl.ds(base + r * (4 * CH), 4 * CH)], mo[r])
            for j in range(CH // 16):
                gi_v[r][pl.ds(j * 16, 16)] = mo[r][pl.ds(j * 16, 16)]
            cps.append(pltpu.async_copy(mn_hbm.at[gi_v[r]], rows[r], sems[r]))
        scps = []
        for r in range(R):
            for j in range(CH // 16):
                si[r][pl.ds(j * 16, 16)] = mo[r][pl.ds(level * CH + j * 16, 16)]
            cps[r].wait()
            scps.append(pltpu.async_copy(rows[r], accum.at[si[r]], sems2[r],
                                         add=True))
        for sc in scps:
            sc.wait()
        return 0

    lax.fori_loop(0, M // R, group, 0)
    plsc.subcore_barrier()
    _copy_out_stripe(ts, accum, msg_hbm.at[c])


def _msg(level):
    return pl.kernel(
        functools.partial(_msg_kernel, level), mesh=_MESH,
        out_type=jax.ShapeDtypeStruct((2, N, D), jnp.float32),
        scratch_types=[
            [pltpu.VMEM((4 * CH,), jnp.int32) for _ in range(R)],
            [pltpu.VMEM((CH,), jnp.int32) for _ in range(R)],
            [pltpu.VMEM((CH,), jnp.int32) for _ in range(R)],
            [pltpu.VMEM((CH, D), jnp.float32) for _ in range(R)],
            pltpu.VMEM_SHARED((NACC, D), jnp.float32),
            [pltpu.SemaphoreType.DMA for _ in range(R)],
            [pltpu.SemaphoreType.DMA for _ in range(R)],
        ],
    )


def kernel(x, edge_index, gate, forward_level, forward_index, Ws, Wt, W_hs,
           b_hs, aggr_W, aggr_b, gru_Wih, gru_bih, gru_Whh, gru_bhh):
    f32 = jnp.float32
    src = jnp.pad(edge_index[0].astype(jnp.int32), (0, EP - E))
    dst = jnp.pad(edge_index[1].astype(jnp.int32), (0, EP - E))
    sdpack = jnp.stack([src.reshape(EP // CH, CH),
                        dst.reshape(EP // CH, CH)], axis=1).reshape(2 * EP)
    gate2 = gate.astype(jnp.int32).reshape(N, 1)
    lv2 = forward_level.astype(jnp.int32).reshape(N, 1)
    ws_p = jnp.pad(Ws.astype(f32), ((0, 2), (0, 0)))
    wt_p = jnp.pad(Wt.astype(f32), ((0, 2), (0, 0)))
    wa = W_hs[:D].astype(f32)
    wb = W_hs[D:].astype(f32)
    b2 = b_hs.reshape(1, D).astype(f32)
    at = aggr_W[:, :D, :].astype(f32)
    abm = aggr_W[:, D:, :].astype(f32)
    ab2 = aggr_b.reshape(4, 1, D).astype(f32)
    bih2 = gru_bih.reshape(4, 1, 3 * D).astype(f32)
    bhh2 = gru_bhh.reshape(4, 1, 3 * D).astype(f32)
    wih = gru_Wih.astype(f32)
    whh = gru_Whh.astype(f32)
    zrows = jnp.zeros((ZR, D), f32)

    xs, xt, pk = _encode(x.astype(jnp.int32), gate2, lv2, ws_p, wt_p)
    ags, agt, midx = _segsum(xs, xt, sdpack, pk.reshape(N), zrows)
    hs, mn = _mix(xs, xt, ags, agt, wa, wb, b2, at, ab2)
    hf = jnp.zeros((N, D), f32)
    for level in range(1, 4):
        msgp = _msg(level)(mn.reshape(4 * N, D), midx, zrows)
        outs = _gru(level, level < 3, msgp, hs, hf, gate2, lv2, wih, bih2,
                    whh, bhh2, at, abm, ab2)
        if level < 3:
            hf, mn = outs
        else:
            hf, = outs
    return hs, hf
